# speedup vs baseline: 1.8168x; 1.8168x over previous
import functools

import jax
import jax.numpy as jnp
from jax.experimental import pallas as pl
from jax.experimental.pallas import tpu as pltpu

N_TOKENS = 16384
HIDDEN = 2048
N_EXPERTS = 64
TOP_K = 2
SCALE = 2.5

TM = 512


def _router_body(x_ref, w_ref, b_ref, logits_ref, packed_ref):
    x = x_ref[...]
    w = w_ref[...]
    logits = jax.lax.dot_general(
        x, w, (((1,), (1,)), ((), ())), preferred_element_type=jnp.float32
    )
    logits_ref[...] = logits

    m = jnp.max(logits, axis=1, keepdims=True)
    z = jnp.exp(logits - m)
    denom = jnp.sum(z, axis=1, keepdims=True)
    scores = z / denom
    s = scores + b_ref[...]

    cols = jax.lax.broadcasted_iota(jnp.int32, s.shape, 1)
    i1 = jnp.argmax(s, axis=1)
    hit1 = cols == i1[:, None]
    s_masked = jnp.where(hit1, -jnp.inf, s)
    i2 = jnp.argmax(s_masked, axis=1)
    hit2 = cols == i2[:, None]
    w1 = jnp.sum(jnp.where(hit1, scores, 0.0), axis=1) * SCALE
    w2 = jnp.sum(jnp.where(hit2, scores, 0.0), axis=1) * SCALE

    pcols = jax.lax.broadcasted_iota(jnp.int32, (x.shape[0], 128), 1)
    out = jnp.where(pcols == 0, w1[:, None], 0.0)
    out = jnp.where(pcols == 1, w2[:, None], out)
    out = jnp.where(pcols == 2, i1[:, None].astype(jnp.float32), out)
    out = jnp.where(pcols == 3, i2[:, None].astype(jnp.float32), out)
    packed_ref[...] = out


def kernel(hidden_states, W, e_score_correction_bias):
    bias = e_score_correction_bias.reshape(1, N_EXPERTS)
    grid = (N_TOKENS // TM,)
    logits, packed = pl.pallas_call(
        _router_body,
        grid=grid,
        in_specs=[
            pl.BlockSpec((TM, HIDDEN), lambda i: (i, 0)),
            pl.BlockSpec((N_EXPERTS, HIDDEN), lambda i: (0, 0)),
            pl.BlockSpec((1, N_EXPERTS), lambda i: (0, 0)),
        ],
        out_specs=[
            pl.BlockSpec((TM, N_EXPERTS), lambda i: (i, 0)),
            pl.BlockSpec((TM, 128), lambda i: (i, 0)),
        ],
        out_shape=[
            jax.ShapeDtypeStruct((N_TOKENS, N_EXPERTS), jnp.float32),
            jax.ShapeDtypeStruct((N_TOKENS, 128), jnp.float32),
        ],
        compiler_params=pltpu.CompilerParams(
            dimension_semantics=("arbitrary",),
        ),
    )(hidden_states, W, bias)
    topk_weights = packed[:, :TOP_K]
    topk_indices = packed[:, TOP_K : 2 * TOP_K].astype(jnp.int32)
    return (logits, topk_weights, topk_indices)


# speedup vs baseline: 1.8677x; 1.0280x over previous
import functools

import jax
import jax.numpy as jnp
from jax import lax
from jax.experimental import pallas as pl
from jax.experimental.pallas import tpu as pltpu
from jax.experimental.pallas import tpu_sc as plsc

N_TOKENS = 16384
HIDDEN = 2048
N_EXPERTS = 64
TOP_K = 2
SCALE = 2.5

TM = 512

NC, NS, L = 2, 16, 16
NW = NC * NS
RPW = N_TOKENS // NW
NBLK = RPW // L


def _mm_body(x_ref, w_ref, lo_ref, lot_ref):
    logits = lax.dot_general(
        x_ref[...], w_ref[...], (((1,), (1,)), ((), ())),
        preferred_element_type=jnp.float32,
    )
    lo_ref[...] = logits
    lot_ref[...] = logits.T


_matmul = pl.pallas_call(
    _mm_body,
    grid=(N_TOKENS // TM,),
    in_specs=[
        pl.BlockSpec((TM, HIDDEN), lambda i: (i, 0)),
        pl.BlockSpec((N_EXPERTS, HIDDEN), lambda i: (0, 0)),
    ],
    out_specs=[
        pl.BlockSpec((TM, N_EXPERTS), lambda i: (i, 0)),
        pl.BlockSpec((N_EXPERTS, TM), lambda i: (0, i)),
    ],
    out_shape=[
        jax.ShapeDtypeStruct((N_TOKENS, N_EXPERTS), jnp.float32),
        jax.ShapeDtypeStruct((N_EXPERTS, N_TOKENS), jnp.float32),
    ],
    compiler_params=pltpu.CompilerParams(
        dimension_semantics=("arbitrary",),
    ),
)


@functools.partial(
    pl.kernel,
    out_type=[
        jax.ShapeDtypeStruct((TOP_K, N_TOKENS), jnp.float32),
        jax.ShapeDtypeStruct((TOP_K, N_TOKENS), jnp.int32),
    ],
    mesh=plsc.VectorSubcoreMesh(
        core_axis_name="c", subcore_axis_name="s",
        num_cores=NC, num_subcores=NS,
    ),
    scratch_types=[
        pltpu.VMEM((N_EXPERTS, RPW), jnp.float32),
        pltpu.VMEM((N_EXPERTS, L), jnp.float32),
        pltpu.VMEM((RPW,), jnp.float32),
        pltpu.VMEM((RPW,), jnp.float32),
        pltpu.VMEM((RPW,), jnp.int32),
        pltpu.VMEM((RPW,), jnp.int32),
    ],
)
def _sc_router(logitsT_hbm, biasb_hbm, twt_hbm, tit_hbm,
               slabT, biasb, w1s, w2s, i1s, i2s):
    wid = lax.axis_index("s") * NC + lax.axis_index("c")
    base = wid * RPW
    pltpu.sync_copy(logitsT_hbm.at[:, pl.ds(base, RPW)], slabT)
    pltpu.sync_copy(biasb_hbm, biasb)

    neg_inf = jnp.full((L,), -jnp.inf, jnp.float32)
    zero_i = jnp.zeros((L,), jnp.int32)

    def block(j, carry):
        sl = pl.ds(j * L, L)
        m = neg_inf
        for e in range(N_EXPERTS):
            m = jnp.maximum(m, slabT[e, sl])
        ssum = jnp.zeros((L,), jnp.float32)
        for e in range(N_EXPERTS):
            z = jnp.exp(slabT[e, sl] - m)
            slabT[e, sl] = z
            ssum = ssum + z
        rinv = 1.0 / ssum
        s1 = neg_inf
        s2 = neg_inf
        w1 = jnp.zeros((L,), jnp.float32)
        w2 = jnp.zeros((L,), jnp.float32)
        i1 = zero_i
        i2 = zero_i
        for e in range(N_EXPERTS):
            sc = slabT[e, sl] * rinv
            s = sc + biasb[e]
            ecol = jnp.full((L,), e, jnp.int32)
            gt1 = s > s1
            gt2 = s > s2
            s2 = jnp.where(gt1, s1, jnp.where(gt2, s, s2))
            w2 = jnp.where(gt1, w1, jnp.where(gt2, sc, w2))
            i2 = jnp.where(gt1, i1, jnp.where(gt2, ecol, i2))
            s1 = jnp.where(gt1, s, s1)
            w1 = jnp.where(gt1, sc, w1)
            i1 = jnp.where(gt1, ecol, i1)
        w1s[sl] = w1 * SCALE
        w2s[sl] = w2 * SCALE
        i1s[sl] = i1
        i2s[sl] = i2
        return carry

    lax.fori_loop(0, NBLK, block, 0)
    pltpu.sync_copy(w1s, twt_hbm.at[0, pl.ds(base, RPW)])
    pltpu.sync_copy(w2s, twt_hbm.at[1, pl.ds(base, RPW)])
    pltpu.sync_copy(i1s, tit_hbm.at[0, pl.ds(base, RPW)])
    pltpu.sync_copy(i2s, tit_hbm.at[1, pl.ds(base, RPW)])


def kernel(hidden_states, W, e_score_correction_bias):
    logits, logitsT = _matmul(hidden_states, W)
    biasb = jnp.broadcast_to(e_score_correction_bias[:, None], (N_EXPERTS, L))
    twt, tit = _sc_router(logitsT, biasb)
    return (logits, twt.T, tit.T)
